# baseline (speedup 1.0000x reference)
import jax
import jax.numpy as jnp
from jax.experimental import pallas as pl
from jax.experimental.pallas import tpu as pltpu
from jax.experimental.pallas import tpu_sc as plsc

_NE = 1024
_ED = 64
_CC = 0.25
_B = 4
_S = 8192
_BS = 2048
_NBLK = _S // _BS
_GRID = _B * _NBLK
_NTOK = _B * _S

_SC_NC = 2
_SC_NS = 16
_TOK_SUB = _NTOK // (_SC_NC * _SC_NS)
_L = 16


def _vq_kernel(x_ref, e_ref, q_ref, idx_ref, cnt_ref, lp_ref):
    x = x_ref[...]
    e = e_ref[...]

    s = jax.lax.dot_general(x, e, (((1,), (1,)), ((), ())),
                            preferred_element_type=jnp.float32)
    xsq = jnp.sum(x * x, axis=1, keepdims=True)
    esq = jnp.sum(e * e, axis=1)[None, :]
    dist = xsq + esq - 2.0 * s

    kiota = jax.lax.broadcasted_iota(jnp.int32, (_BS, _NE), 1)
    dmin = jnp.min(dist, axis=1, keepdims=True)
    idx = jnp.min(jnp.where(dist == dmin, kiota, _NE), axis=1)
    idx_ref[...] = idx[:, None]

    oh = (kiota == idx[:, None]).astype(jnp.float32)
    q = jnp.dot(oh, e, preferred_element_type=jnp.float32)
    q_ref[...] = q

    cnt_ref[0, 0] = jnp.sum(oh, axis=0)
    lp_ref[0, 0] = jnp.broadcast_to(jnp.sum(dmin, axis=0), (_NE,))


_TCH = 64
_NCH = _TOK_SUB // _TCH


def _sc_oh_kernel(zeros_hbm, idx_hbm, out_hbm, buf, idxv, sem):
    c = jax.lax.axis_index("c")
    s = jax.lax.axis_index("s")
    w = c * _SC_NS + s

    pltpu.sync_copy(zeros_hbm, buf)
    base_tok = pl.multiple_of(w * _TOK_SUB, _TOK_SUB)
    pltpu.sync_copy(idx_hbm.at[pl.ds(base_tok, _TOK_SUB)], idxv)

    lane = jax.lax.iota(jnp.int32, _L)

    def _chunk(t, carry):
        def _scat(val0):
            def _body(g, carry2):
                toff = pl.multiple_of(t * _TCH + g * _L, _L)
                v16 = idxv[pl.ds(toff, _L)]
                for j in range(_L):
                    vj = v16[j]
                    row = g * _L + j
                    bs = (vj // _L) * _L
                    pat = jnp.where(lane == vj % _L, val0, 0.0)
                    buf[pl.ds(pl.multiple_of(row * _NE + bs, _L), _L)] = pat
                return carry2
            jax.lax.fori_loop(0, _TCH // _L, _body, 0)

        _scat(jnp.float32(1.0))
        dst = pl.multiple_of((base_tok + t * _TCH) * _NE, _TCH * _NE)
        pltpu.sync_copy(buf, out_hbm.at[pl.ds(dst, _TCH * _NE)])
        _scat(jnp.float32(0.0))
        return carry

    jax.lax.fori_loop(0, _NCH, _chunk, 0)


def _fin_kernel(cnt_ref, lp_ref, loss_ref, perp_ref):
    cnt = jnp.sum(cnt_ref[...], axis=0, keepdims=True)
    p = cnt * (1.0 / _NTOK)
    perp_ref[...] = jnp.exp(-jnp.sum(p * jnp.log(p + 1e-10), keepdims=True))
    lsum = jnp.sum(lp_ref[...][:, 0:1], keepdims=True)
    loss_ref[...] = lsum * (_CC / (_NTOK * _ED))


def kernel(inputs, embed):
    x = jnp.transpose(inputs, (0, 2, 3, 4, 1)).reshape(_NTOK, _ED)

    q, idx, cnt, lp = pl.pallas_call(
        _vq_kernel,
        grid=(_GRID,),
        in_specs=[
            pl.BlockSpec((_BS, _ED), lambda g: (g, 0)),
            pl.BlockSpec((_NE, _ED), lambda g: (0, 0)),
        ],
        out_specs=[
            pl.BlockSpec((_BS, _ED), lambda g: (g, 0)),
            pl.BlockSpec((_BS, 1), lambda g: (g, 0)),
            pl.BlockSpec((1, 1, _NE), lambda g: (g, 0, 0)),
            pl.BlockSpec((1, 1, _NE), lambda g: (g, 0, 0)),
        ],
        out_shape=[
            jax.ShapeDtypeStruct((_NTOK, _ED), jnp.float32),
            jax.ShapeDtypeStruct((_NTOK, 1), jnp.int32),
            jax.ShapeDtypeStruct((_GRID, 1, _NE), jnp.float32),
            jax.ShapeDtypeStruct((_GRID, 1, _NE), jnp.float32),
        ],
        compiler_params=pltpu.CompilerParams(
            dimension_semantics=("parallel",),
        ),
    )(x, embed)

    oh = pl.kernel(
        _sc_oh_kernel,
        out_type=jax.ShapeDtypeStruct((_NTOK * _NE,), jnp.float32),
        mesh=plsc.VectorSubcoreMesh(core_axis_name="c", subcore_axis_name="s"),
        scratch_types=[
            pltpu.VMEM((_TCH * _NE,), jnp.float32),
            pltpu.VMEM((_TOK_SUB,), jnp.int32),
            pltpu.SemaphoreType.DMA,
        ],
    )(
        jnp.zeros((_TCH * _NE,), jnp.float32),
        idx.reshape(_NTOK),
    )

    loss, perp = pl.pallas_call(
        _fin_kernel,
        out_specs=[
            pl.BlockSpec((1, 1), lambda: (0, 0)),
            pl.BlockSpec((1, 1), lambda: (0, 0)),
        ],
        out_shape=[
            jax.ShapeDtypeStruct((1, 1), jnp.float32),
            jax.ShapeDtypeStruct((1, 1), jnp.float32),
        ],
    )(cnt.reshape(_GRID, _NE), lp.reshape(_GRID, _NE))

    quantized_st = jnp.transpose(q.reshape(_B, 8, 32, 32, _ED), (0, 4, 1, 2, 3))
    oh_r = jnp.transpose(oh.reshape(_B, 8, 32, 32, _NE), (0, 4, 1, 2, 3))
    encoding_indices = idx.reshape(_NTOK)
    return (loss[0, 0], quantized_st, perp[0, 0], oh_r, encoding_indices)


# speedup vs baseline: 1.0063x; 1.0063x over previous
import jax
import jax.numpy as jnp
from jax.experimental import pallas as pl
from jax.experimental.pallas import tpu as pltpu
from jax.experimental.pallas import tpu_sc as plsc

_NE = 1024
_ED = 64
_CC = 0.25
_B = 4
_S = 8192
_BS = 4096
_NBLK = _S // _BS
_GRID = _B * _NBLK
_NTOK = _B * _S

_SC_NC = 2
_SC_NS = 16
_TOK_SUB = _NTOK // (_SC_NC * _SC_NS)
_L = 16


def _vq_kernel(x_ref, e_ref, q_ref, idx_ref, cnt_ref, lp_ref):
    x = x_ref[...]
    e = e_ref[...]

    s = jax.lax.dot_general(x, e, (((1,), (1,)), ((), ())),
                            preferred_element_type=jnp.float32)
    xsq = jnp.sum(x * x, axis=1, keepdims=True)
    esq = jnp.sum(e * e, axis=1)[None, :]
    dist = xsq + esq - 2.0 * s

    kiota = jax.lax.broadcasted_iota(jnp.int32, (_BS, _NE), 1)
    dmin = jnp.min(dist, axis=1, keepdims=True)
    idx = jnp.min(jnp.where(dist == dmin, kiota, _NE), axis=1)
    idx_ref[...] = idx[:, None]

    oh = (kiota == idx[:, None]).astype(jnp.float32)
    q = jnp.dot(oh, e, preferred_element_type=jnp.float32)
    q_ref[...] = q

    cnt_ref[0, 0] = jnp.sum(oh, axis=0)
    lp_ref[0, 0] = jnp.broadcast_to(jnp.sum(dmin, axis=0), (_NE,))


_TCH = 64
_NCH = _TOK_SUB // _TCH


def _sc_oh_kernel(zeros_hbm, idx_hbm, out_hbm, buf, idxv, sem):
    c = jax.lax.axis_index("c")
    s = jax.lax.axis_index("s")
    w = c * _SC_NS + s

    pltpu.sync_copy(zeros_hbm, buf)
    base_tok = pl.multiple_of(w * _TOK_SUB, _TOK_SUB)
    pltpu.sync_copy(idx_hbm.at[pl.ds(base_tok, _TOK_SUB)], idxv)

    lane = jax.lax.iota(jnp.int32, _L)

    def _chunk(t, carry):
        def _scat(val0):
            def _body(g, carry2):
                toff = pl.multiple_of(t * _TCH + g * _L, _L)
                v16 = idxv[pl.ds(toff, _L)]
                for j in range(_L):
                    vj = v16[j]
                    row = g * _L + j
                    bs = (vj // _L) * _L
                    pat = jnp.where(lane == vj % _L, val0, 0.0)
                    buf[pl.ds(pl.multiple_of(row * _NE + bs, _L), _L)] = pat
                return carry2
            jax.lax.fori_loop(0, _TCH // _L, _body, 0)

        _scat(jnp.float32(1.0))
        dst = pl.multiple_of((base_tok + t * _TCH) * _NE, _TCH * _NE)
        pltpu.sync_copy(buf, out_hbm.at[pl.ds(dst, _TCH * _NE)])
        _scat(jnp.float32(0.0))
        return carry

    jax.lax.fori_loop(0, _NCH, _chunk, 0)


def _fin_kernel(cnt_ref, lp_ref, loss_ref, perp_ref):
    cnt = jnp.sum(cnt_ref[...], axis=0, keepdims=True)
    p = cnt * (1.0 / _NTOK)
    perp_ref[...] = jnp.exp(-jnp.sum(p * jnp.log(p + 1e-10), keepdims=True))
    lsum = jnp.sum(lp_ref[...][:, 0:1], keepdims=True)
    loss_ref[...] = lsum * (_CC / (_NTOK * _ED))


def kernel(inputs, embed):
    x = jnp.transpose(inputs, (0, 2, 3, 4, 1)).reshape(_NTOK, _ED)

    q, idx, cnt, lp = pl.pallas_call(
        _vq_kernel,
        grid=(_GRID,),
        in_specs=[
            pl.BlockSpec((_BS, _ED), lambda g: (g, 0)),
            pl.BlockSpec((_NE, _ED), lambda g: (0, 0)),
        ],
        out_specs=[
            pl.BlockSpec((_BS, _ED), lambda g: (g, 0)),
            pl.BlockSpec((_BS, 1), lambda g: (g, 0)),
            pl.BlockSpec((1, 1, _NE), lambda g: (g, 0, 0)),
            pl.BlockSpec((1, 1, _NE), lambda g: (g, 0, 0)),
        ],
        out_shape=[
            jax.ShapeDtypeStruct((_NTOK, _ED), jnp.float32),
            jax.ShapeDtypeStruct((_NTOK, 1), jnp.int32),
            jax.ShapeDtypeStruct((_GRID, 1, _NE), jnp.float32),
            jax.ShapeDtypeStruct((_GRID, 1, _NE), jnp.float32),
        ],
        compiler_params=pltpu.CompilerParams(
            dimension_semantics=("parallel",),
        ),
    )(x, embed)

    oh = pl.kernel(
        _sc_oh_kernel,
        out_type=jax.ShapeDtypeStruct((_NTOK * _NE,), jnp.float32),
        mesh=plsc.VectorSubcoreMesh(core_axis_name="c", subcore_axis_name="s"),
        scratch_types=[
            pltpu.VMEM((_TCH * _NE,), jnp.float32),
            pltpu.VMEM((_TOK_SUB,), jnp.int32),
            pltpu.SemaphoreType.DMA,
        ],
    )(
        jnp.zeros((_TCH * _NE,), jnp.float32),
        idx.reshape(_NTOK),
    )

    loss, perp = pl.pallas_call(
        _fin_kernel,
        out_specs=[
            pl.BlockSpec((1, 1), lambda: (0, 0)),
            pl.BlockSpec((1, 1), lambda: (0, 0)),
        ],
        out_shape=[
            jax.ShapeDtypeStruct((1, 1), jnp.float32),
            jax.ShapeDtypeStruct((1, 1), jnp.float32),
        ],
    )(cnt.reshape(_GRID, _NE), lp.reshape(_GRID, _NE))

    quantized_st = jnp.transpose(q.reshape(_B, 8, 32, 32, _ED), (0, 4, 1, 2, 3))
    oh_r = jnp.transpose(oh.reshape(_B, 8, 32, 32, _NE), (0, 4, 1, 2, 3))
    encoding_indices = idx.reshape(_NTOK)
    return (loss[0, 0], quantized_st, perp[0, 0], oh_r, encoding_indices)


# speedup vs baseline: 2.7592x; 2.7419x over previous
import jax
import jax.numpy as jnp
from jax.experimental import pallas as pl
from jax.experimental.pallas import tpu as pltpu

_NE = 1024
_ED = 64
_CC = 0.25
_B = 4
_S = 8192
_BS = 2048
_GRID = _B * _S // _BS
_NTOK = _B * _S


def _vq_kernel(x_ref, e_ref, oh_ref, q_ref, idx_ref, cnt_ref, lp_ref):
    x = x_ref[...]
    e = e_ref[...]

    s = jax.lax.dot_general(x, e, (((1,), (1,)), ((), ())),
                            preferred_element_type=jnp.float32)
    xsq = jnp.sum(x * x, axis=1, keepdims=True)
    esq = jnp.sum(e * e, axis=1)[None, :]
    dist = xsq + esq - 2.0 * s

    kiota = jax.lax.broadcasted_iota(jnp.int32, (_BS, _NE), 1)
    dmin = jnp.min(dist, axis=1, keepdims=True)
    idx = jnp.min(jnp.where(dist == dmin, kiota, _NE), axis=1)
    idx_ref[...] = idx[:, None]

    oh = (kiota == idx[:, None]).astype(jnp.float32)
    oh_ref[...] = oh
    q = jnp.dot(oh, e, preferred_element_type=jnp.float32)
    q_ref[...] = q

    cnt_ref[0, 0] = jnp.sum(oh, axis=0)
    lp_ref[0, 0] = jnp.broadcast_to(jnp.sum(dmin, axis=0), (_NE,))


def _fin_kernel(cnt_ref, lp_ref, loss_ref, perp_ref):
    cnt = jnp.sum(cnt_ref[...], axis=0, keepdims=True)
    p = cnt * (1.0 / _NTOK)
    perp_ref[...] = jnp.exp(-jnp.sum(p * jnp.log(p + 1e-10), keepdims=True))
    lsum = jnp.sum(lp_ref[...][:, 0:1], keepdims=True)
    loss_ref[...] = lsum * (_CC / (_NTOK * _ED))


def kernel(inputs, embed):
    x = jnp.transpose(inputs, (0, 2, 3, 4, 1)).reshape(_NTOK, _ED)

    oh, q, idx, cnt, lp = pl.pallas_call(
        _vq_kernel,
        grid=(_GRID,),
        in_specs=[
            pl.BlockSpec((_BS, _ED), lambda g: (g, 0)),
            pl.BlockSpec((_NE, _ED), lambda g: (0, 0)),
        ],
        out_specs=[
            pl.BlockSpec((_BS, _NE), lambda g: (g, 0)),
            pl.BlockSpec((_BS, _ED), lambda g: (g, 0)),
            pl.BlockSpec((_BS, 1), lambda g: (g, 0)),
            pl.BlockSpec((1, 1, _NE), lambda g: (g, 0, 0)),
            pl.BlockSpec((1, 1, _NE), lambda g: (g, 0, 0)),
        ],
        out_shape=[
            jax.ShapeDtypeStruct((_NTOK, _NE), jnp.float32),
            jax.ShapeDtypeStruct((_NTOK, _ED), jnp.float32),
            jax.ShapeDtypeStruct((_NTOK, 1), jnp.int32),
            jax.ShapeDtypeStruct((_GRID, 1, _NE), jnp.float32),
            jax.ShapeDtypeStruct((_GRID, 1, _NE), jnp.float32),
        ],
        compiler_params=pltpu.CompilerParams(
            dimension_semantics=("parallel",),
        ),
    )(x, embed)

    loss, perp = pl.pallas_call(
        _fin_kernel,
        out_specs=[
            pl.BlockSpec((1, 1), lambda: (0, 0)),
            pl.BlockSpec((1, 1), lambda: (0, 0)),
        ],
        out_shape=[
            jax.ShapeDtypeStruct((1, 1), jnp.float32),
            jax.ShapeDtypeStruct((1, 1), jnp.float32),
        ],
    )(cnt.reshape(_GRID, _NE), lp.reshape(_GRID, _NE))

    quantized_st = jnp.transpose(q.reshape(_B, 8, 32, 32, _ED), (0, 4, 1, 2, 3))
    oh_r = jnp.transpose(oh.reshape(_B, 8, 32, 32, _NE), (0, 4, 1, 2, 3))
    encoding_indices = idx.reshape(_NTOK)
    return (loss[0, 0], quantized_st, perp[0, 0], oh_r, encoding_indices)


# speedup vs baseline: 2.9504x; 1.0693x over previous
import jax
import jax.numpy as jnp
from jax.experimental import pallas as pl
from jax.experimental.pallas import tpu as pltpu

_NE = 1024
_ED = 64
_CC = 0.25
_B = 4
_S = 8192
_BS = 2048
_GRID = _B * _S // _BS
_NTOK = _B * _S


def _vq_kernel(x_ref, e_ref, oh_ref, q_ref, idx_ref, cnt_ref, lp_ref):
    x = x_ref[...]
    e = e_ref[...]

    s = jax.lax.dot_general(x, e, (((1,), (1,)), ((), ())),
                            preferred_element_type=jnp.float32)
    xsq = jnp.sum(x * x, axis=1, keepdims=True)
    esq = jnp.sum(e * e, axis=1)[None, :]
    dist = xsq + esq - 2.0 * s

    kiota = jax.lax.broadcasted_iota(jnp.int32, (_BS, _NE), 1)
    dmin = jnp.min(dist, axis=1, keepdims=True)
    idx = jnp.min(jnp.where(dist == dmin, kiota, _NE), axis=1)
    idx_ref[...] = idx[:, None]

    oh = (kiota == idx[:, None]).astype(jnp.float32)
    oh_ref[...] = oh
    q = jnp.dot(oh, e, preferred_element_type=jnp.float32)
    q_ref[...] = q

    ones = jnp.ones((1, _BS), jnp.float32)
    cnt_ref[0, 0] = jnp.dot(ones, oh,
                            preferred_element_type=jnp.float32)[0]
    lp_ref[0, 0] = jnp.broadcast_to(jnp.sum(dmin, axis=0), (_NE,))


def _fin_kernel(cnt_ref, lp_ref, loss_ref, perp_ref):
    cnt = jnp.sum(cnt_ref[...], axis=0, keepdims=True)
    p = cnt * (1.0 / _NTOK)
    perp_ref[...] = jnp.exp(-jnp.sum(p * jnp.log(p + 1e-10), keepdims=True))
    lsum = jnp.sum(lp_ref[...][:, 0:1], keepdims=True)
    loss_ref[...] = lsum * (_CC / (_NTOK * _ED))


def kernel(inputs, embed):
    x = jnp.transpose(inputs, (0, 2, 3, 4, 1)).reshape(_NTOK, _ED)

    oh, q, idx, cnt, lp = pl.pallas_call(
        _vq_kernel,
        grid=(_GRID,),
        in_specs=[
            pl.BlockSpec((_BS, _ED), lambda g: (g, 0)),
            pl.BlockSpec((_NE, _ED), lambda g: (0, 0)),
        ],
        out_specs=[
            pl.BlockSpec((_BS, _NE), lambda g: (g, 0)),
            pl.BlockSpec((_BS, _ED), lambda g: (g, 0)),
            pl.BlockSpec((_BS, 1), lambda g: (g, 0)),
            pl.BlockSpec((1, 1, _NE), lambda g: (g, 0, 0)),
            pl.BlockSpec((1, 1, _NE), lambda g: (g, 0, 0)),
        ],
        out_shape=[
            jax.ShapeDtypeStruct((_NTOK, _NE), jnp.float32),
            jax.ShapeDtypeStruct((_NTOK, _ED), jnp.float32),
            jax.ShapeDtypeStruct((_NTOK, 1), jnp.int32),
            jax.ShapeDtypeStruct((_GRID, 1, _NE), jnp.float32),
            jax.ShapeDtypeStruct((_GRID, 1, _NE), jnp.float32),
        ],
        compiler_params=pltpu.CompilerParams(
            dimension_semantics=("parallel",),
        ),
    )(x, embed)

    loss, perp = pl.pallas_call(
        _fin_kernel,
        out_specs=[
            pl.BlockSpec((1, 1), lambda: (0, 0)),
            pl.BlockSpec((1, 1), lambda: (0, 0)),
        ],
        out_shape=[
            jax.ShapeDtypeStruct((1, 1), jnp.float32),
            jax.ShapeDtypeStruct((1, 1), jnp.float32),
        ],
    )(cnt.reshape(_GRID, _NE), lp.reshape(_GRID, _NE))

    quantized_st = jnp.transpose(q.reshape(_B, 8, 32, 32, _ED), (0, 4, 1, 2, 3))
    oh_r = jnp.transpose(oh.reshape(_B, 8, 32, 32, _NE), (0, 4, 1, 2, 3))
    encoding_indices = idx.reshape(_NTOK)
    return (loss[0, 0], quantized_st, perp[0, 0], oh_r, encoding_indices)
